# baseline (speedup 1.0000x reference)
import functools

import jax
import jax.numpy as jnp
from jax import lax
from jax.experimental import pallas as pl
from jax.experimental.pallas import tpu as pltpu
from jax.experimental.pallas import tpu_sc as plsc

N = 10000
NP = 10240
E = 320000
EB = 2 * E
NW = 32
G_BI = 160
EB_PAD = NW * G_BI * 128
G_E = 80
E_PAD = NW * G_E * 128
NS = 16
RPT = NP // NS
NHR = NP // 128
HALF = NP // 2

f32 = jnp.float32


BLK = 256
BLK_E = 512


def _tca_body(nf, w, b, wl, h0_ref, p0_ref):
    h0 = jnp.dot(nf[...], w[...], preferred_element_type=f32) + b[...]
    h0_ref[...] = h0
    p0_ref[...] = jnp.dot(h0, wl[...], preferred_element_type=f32)


_tc_a = pl.pallas_call(
    _tca_body,
    grid=(NP // BLK,),
    in_specs=[
        pl.BlockSpec((BLK, 144), lambda i: (i, 0)),
        pl.BlockSpec((144, 128), lambda i: (0, 0)),
        pl.BlockSpec((1, 128), lambda i: (0, 0)),
        pl.BlockSpec((128, 128), lambda i: (0, 0)),
    ],
    out_specs=[
        pl.BlockSpec((BLK, 128), lambda i: (i, 0)),
        pl.BlockSpec((BLK, 128), lambda i: (i, 0)),
    ],
    out_shape=[
        jax.ShapeDtypeStruct((NP, 128), f32),
        jax.ShapeDtypeStruct((NP, 128), f32),
    ],
)


def _deg_col(degacc_ref):
    d2 = degacc_ref[0, 0] + degacc_ref[1, 0]
    a_const = (lax.broadcasted_iota(jnp.int32, (BLK, 2), 0) // 128
               == lax.broadcasted_iota(jnp.int32, (BLK, 2), 1)).astype(f32)
    sel = (lax.broadcasted_iota(jnp.int32, (BLK, 128), 0) % 128
           == lax.broadcasted_iota(jnp.int32, (BLK, 128), 1)).astype(f32)
    spread = jnp.dot(a_const, d2, preferred_element_type=f32)
    deg = jnp.sum(spread * sel, axis=1, keepdims=True)
    return jnp.maximum(deg, 1.0)


def _tcb_body(acc_ref, degacc_ref, h0_ref, wr, b, wl1, h1_ref, p1_ref, degb_ref):
    acc = acc_ref[0] + acc_ref[1]
    deg = _deg_col(degacc_ref)
    h1 = jnp.maximum(
        acc / deg + jnp.dot(h0_ref[...], wr[...], preferred_element_type=f32)
        + b[...], 0.0)
    h1_ref[...] = h1
    p1_ref[...] = jnp.dot(h1, wl1[...], preferred_element_type=f32)
    degb_ref[...] = jnp.broadcast_to(deg, (BLK, 128))


_tc_b = pl.pallas_call(
    _tcb_body,
    grid=(NP // BLK,),
    in_specs=[
        pl.BlockSpec((2, BLK, 128), lambda i: (0, i, 0)),
        pl.BlockSpec((2, 1, 2, 128), lambda i: (0, i, 0, 0)),
        pl.BlockSpec((BLK, 128), lambda i: (i, 0)),
        pl.BlockSpec((128, 128), lambda i: (0, 0)),
        pl.BlockSpec((1, 128), lambda i: (0, 0)),
        pl.BlockSpec((128, 128), lambda i: (0, 0)),
    ],
    out_specs=[
        pl.BlockSpec((BLK, 128), lambda i: (i, 0)),
        pl.BlockSpec((BLK, 128), lambda i: (i, 0)),
        pl.BlockSpec((BLK, 128), lambda i: (i, 0)),
    ],
    out_shape=[
        jax.ShapeDtypeStruct((NP, 128), f32),
        jax.ShapeDtypeStruct((NP, 128), f32),
        jax.ShapeDtypeStruct((NP, 128), f32),
    ],
)


def _tcc_body(acc_ref, h1_ref, degb_ref, wr, b, nw1, nb1, nw2, nb2, wa, wb,
              eb1, atab_ref, btab_ref, ph_ref):
    acc = acc_ref[0] + acc_ref[1]
    h2 = (acc / degb_ref[...]
          + jnp.dot(h1_ref[...], wr[...], preferred_element_type=f32) + b[...])
    atab_ref[...] = jnp.dot(h2, wa[...], preferred_element_type=f32) + eb1[...]
    btab_ref[...] = jnp.dot(h2, wb[...], preferred_element_type=f32)
    t = jnp.maximum(jnp.dot(h2, nw1[...], preferred_element_type=f32) + nb1[...], 0.0)
    ph = jnp.dot(t, nw2[...], preferred_element_type=f32) + nb2[...]
    ph_ref[...] = ph[:, :8]


_tc_c = pl.pallas_call(
    _tcc_body,
    grid=(NP // BLK,),
    in_specs=[
        pl.BlockSpec((2, BLK, 128), lambda i: (0, i, 0)),
        pl.BlockSpec((BLK, 128), lambda i: (i, 0)),
        pl.BlockSpec((BLK, 128), lambda i: (i, 0)),
        pl.BlockSpec((128, 128), lambda i: (0, 0)),
        pl.BlockSpec((1, 128), lambda i: (0, 0)),
        pl.BlockSpec((128, 128), lambda i: (0, 0)),
        pl.BlockSpec((1, 128), lambda i: (0, 0)),
        pl.BlockSpec((128, 128), lambda i: (0, 0)),
        pl.BlockSpec((1, 128), lambda i: (0, 0)),
        pl.BlockSpec((128, 128), lambda i: (0, 0)),
        pl.BlockSpec((128, 128), lambda i: (0, 0)),
        pl.BlockSpec((1, 128), lambda i: (0, 0)),
    ],
    out_specs=[
        pl.BlockSpec((BLK, 128), lambda i: (i, 0)),
        pl.BlockSpec((BLK, 128), lambda i: (i, 0)),
        pl.BlockSpec((BLK, 8), lambda i: (i, 0)),
    ],
    out_shape=[
        jax.ShapeDtypeStruct((NP, 128), f32),
        jax.ShapeDtypeStruct((NP, 128), f32),
        jax.ShapeDtypeStruct((NP, 8), f32),
    ],
)


def _tcd_body(sa_ref, ef_ref, wc, w2, b2, q_ref):
    hidden = (sa_ref[...]
              + jnp.dot(ef_ref[...], wc[...], preferred_element_type=f32))
    q = jnp.dot(jnp.maximum(hidden, 0.0), w2[...], preferred_element_type=f32) + b2[...]
    q_ref[...] = q[:, :8]


_tc_d = pl.pallas_call(
    _tcd_body,
    grid=(E_PAD // BLK_E,),
    in_specs=[
        pl.BlockSpec((BLK_E, 128), lambda i: (i, 0)),
        pl.BlockSpec((BLK_E, 32), lambda i: (i, 0)),
        pl.BlockSpec((32, 128), lambda i: (0, 0)),
        pl.BlockSpec((128, 128), lambda i: (0, 0)),
        pl.BlockSpec((1, 128), lambda i: (0, 0)),
    ],
    out_specs=pl.BlockSpec((BLK_E, 8), lambda i: (i, 0)),
    out_shape=jax.ShapeDtypeStruct((E_PAD, 8), f32),
)


_MESH = dict(core_axis_name="c", subcore_axis_name="s")


ACC_RPT = NP // NS
CH = 64
IB2 = 32
DEPTH = 4


def _make_sc_scatter(G2):
    @functools.partial(
        pl.kernel,
        out_type=jax.ShapeDtypeStruct((2, NP, 128), f32),
        mesh=plsc.VectorSubcoreMesh(**_MESH),
        scratch_types=(
            [pltpu.VMEM((IB2, CH), jnp.int32),
             pltpu.VMEM((IB2, CH), jnp.int32)]
            + [pltpu.VMEM((CH, 128), f32)] * DEPTH
            + [pltpu.VMEM_SHARED((NP, 128), f32)]
            + [pltpu.SemaphoreType.DMA] * (2 * DEPTH)
        ),
    )
    def sc_scatter(table, sidx, didx, zeros, out, sidx_v, didx_v, *scr):
        rbufs = scr[:DEPTH]
        acc_sh = scr[DEPTH]
        gsems = scr[DEPTH + 1:DEPTH + 1 + DEPTH]
        ssems = scr[DEPTH + 1 + DEPTH:]
        c = lax.axis_index("c")
        s = lax.axis_index("s")
        wid = s * 2 + c
        pltpu.sync_copy(zeros.at[pl.ds(s * ACC_RPT, ACC_RPT)],
                        acc_sh.at[pl.ds(s * ACC_RPT, ACC_RPT)])
        plsc.subcore_barrier()

        def blk_body(j, carry):
            pltpu.sync_copy(sidx.at[wid, pl.ds(j * IB2, IB2)], sidx_v)
            pltpu.sync_copy(didx.at[wid, pl.ds(j * IB2, IB2)], didx_v)

            def body(i, carry2):
                g = i * DEPTH
                hs = [pltpu.async_copy(table.at[sidx_v.at[g + k]], rbufs[k],
                                       gsems[k])
                      for k in range(DEPTH)]
                ws = []
                for k in range(DEPTH):
                    hs[k].wait()
                    ws.append(pltpu.async_copy(
                        rbufs[k], acc_sh.at[didx_v.at[g + k]], ssems[k],
                        add=True))
                for w in ws:
                    w.wait()
                return carry2

            lax.fori_loop(0, IB2 // DEPTH, body, 0)
            return carry

        lax.fori_loop(0, G2 // IB2, blk_body, 0)
        plsc.subcore_barrier()
        pltpu.sync_copy(acc_sh.at[pl.ds(s * ACC_RPT, ACC_RPT)],
                        out.at[c, pl.ds(s * ACC_RPT, ACC_RPT)])

    return sc_scatter


_sc_scatter = _make_sc_scatter(G_BI * 128 // CH)


@functools.partial(
    pl.kernel,
    out_type=jax.ShapeDtypeStruct((2, NHR, 128), f32),
    mesh=plsc.VectorSubcoreMesh(**_MESH),
    compiler_params=pltpu.CompilerParams(needs_layout_passes=False),
    scratch_types=[
        pltpu.VMEM((G_BI, 128), jnp.int32),
        pltpu.VMEM((16 * HALF,), f32),
        pltpu.VMEM((NHR, 128), f32),
        pltpu.VMEM((1, NHR), jnp.int32),
        pltpu.VMEM_SHARED((NHR, 128), f32),
        pltpu.SemaphoreType.DMA,
    ],
)
def _sc_deg(didx, rowids, zeros, out, didx_v, hist_v, deg_v, rid_v, acc_sh, sem):
    c = lax.axis_index("c")
    s = lax.axis_index("s")
    wid = s * 2 + c
    pltpu.sync_copy(didx.at[wid], didx_v)
    pltpu.sync_copy(rowids, rid_v)

    @pl.when(s < NHR // 8)
    def _zero():
        pltpu.sync_copy(zeros.at[pl.ds(s * 8, 8)], acc_sh.at[pl.ds(s * 8, 8)])

    lane_off = lax.iota(jnp.int32, 16) * HALF
    ones16 = jnp.ones((16,), f32)
    zeros16 = jnp.zeros((16,), f32)

    for half in range(2):
        lo = half * HALF

        def zero_body(j, carry):
            hist_v[pl.ds(j * 16, 16)] = zeros16
            return carry

        lax.fori_loop(0, 16 * (HALF // 16), zero_body, 0)

        def count_body(g, carry):
            for k in range(8):
                idx16 = didx_v[g, pl.ds(k * 16, 16)]
                m = (idx16 >= lo) & (idx16 < lo + HALF)
                idxc = jnp.where(m, idx16 - lo, 0) + lane_off
                plsc.addupdate_scatter(hist_v, [idxc], ones16, mask=m)
            return carry

        lax.fori_loop(0, G_BI, count_body, 0)

        def fold_body(j, carry):
            t = hist_v[pl.ds(j * 16, 16)]
            for r in range(1, 16):
                t = t + hist_v[pl.ds(r * HALF + j * 16, 16)]
            flat = half * HALF + j * 16
            deg_v[flat // 128, pl.ds(flat % 128, 16)] = t
            return carry

        lax.fori_loop(0, HALF // 16, fold_body, 0)

    plsc.subcore_barrier()
    pltpu.sync_copy(deg_v, acc_sh.at[rid_v.at[0]], add=True)
    plsc.subcore_barrier()

    @pl.when(s < NHR // 8)
    def _writeback():
        pltpu.sync_copy(acc_sh.at[pl.ds(s * 8, 8)], out.at[c, pl.ds(s * 8, 8)])


DEPTH_E = 4


@functools.partial(
    pl.kernel,
    out_type=jax.ShapeDtypeStruct((E_PAD, 128), f32),
    mesh=plsc.VectorSubcoreMesh(**_MESH),
    scratch_types=(
        [pltpu.VMEM((G_E, 128), jnp.int32),
         pltpu.VMEM((G_E, 128), jnp.int32)]
        + [pltpu.VMEM((128, 128), f32)] * DEPTH_E
        + [pltpu.SemaphoreType.DMA] * (3 * DEPTH_E)
    ),
)
def _sc_edge_gather(atab, btab, sidx, didx, outa, sidx_v, didx_v, *scr):
    rbufs = scr[:DEPTH_E]
    gsems = scr[DEPTH_E:2 * DEPTH_E]
    asems = scr[2 * DEPTH_E:3 * DEPTH_E]
    wsems = scr[3 * DEPTH_E:]
    c = lax.axis_index("c")
    s = lax.axis_index("s")
    wid = s * 2 + c
    pltpu.sync_copy(sidx.at[wid], sidx_v)
    pltpu.sync_copy(didx.at[wid], didx_v)
    base = wid * (G_E * 128)

    def body(i, carry):
        g = i * DEPTH_E
        hs = [pltpu.async_copy(atab.at[sidx_v.at[g + k]], rbufs[k], gsems[k])
              for k in range(DEPTH_E)]
        ads = []
        for k in range(DEPTH_E):
            hs[k].wait()
            ads.append(pltpu.async_copy(btab.at[didx_v.at[g + k]], rbufs[k],
                                        asems[k], add=True))
        ws = []
        for k in range(DEPTH_E):
            ads[k].wait()
            ws.append(pltpu.async_copy(
                rbufs[k], outa.at[pl.ds(base + (g + k) * 128, 128)], wsems[k]))
        for w in ws:
            w.wait()
        return carry

    lax.fori_loop(0, G_E // DEPTH_E, body, 0)


def kernel(edge_index, node_static, edge_static, p_obs, q_obs, p_mask, q_mask, params):
    pr = params
    ei = edge_index.astype(jnp.int32)
    e0, e1 = ei[0], ei[1]

    nf = jnp.concatenate(
        [node_static, p_obs[:, None], p_mask[:, None].astype(f32)], axis=1)
    nf = jnp.pad(nf, ((0, NP - N), (0, 144 - nf.shape[1])))
    encW = jnp.pad(pr['enc_W'], ((0, 144 - pr['enc_W'].shape[0]), (0, 0)))

    src = jnp.concatenate([e0, e1])
    dst = jnp.concatenate([e1, e0])
    pad_bi = jnp.full((EB_PAD - EB,), N, jnp.int32)
    src_pad = jnp.concatenate([src, pad_bi])
    dst_pad = jnp.concatenate([dst, pad_bi])
    srcp = src_pad.reshape(NW, G_BI * 128 // CH, CH)
    dstp = dst_pad.reshape(NW, G_BI * 128 // CH, CH)
    pad_e = jnp.full((E_PAD - E,), N, jnp.int32)
    sp = jnp.concatenate([e0, pad_e]).reshape(NW, G_E, 128)
    dp = jnp.concatenate([e1, pad_e]).reshape(NW, G_E, 128)

    zeros_acc = jnp.zeros((NP, 128), f32)
    zeros_deg = jnp.zeros((NHR, 128), f32)
    rowids = jnp.arange(NHR, dtype=jnp.int32)[None, :]

    degacc = _sc_deg(dst_pad.reshape(NW, G_BI, 128), rowids, zeros_deg)
    h0, p0 = _tc_a(nf, encW, pr['enc_b'][None], pr['sage0_Wl'])
    acc0 = _sc_scatter(p0, srcp, dstp, zeros_acc)
    h1, p1, degb = _tc_b(acc0, degacc.reshape(2, NHR // 2, 2, 128), h0,
                         pr['sage0_Wr'], pr['sage0_b'][None], pr['sage1_Wl'])
    acc1 = _sc_scatter(p1, srcp, dstp, zeros_acc)
    ehW1 = pr['eh_W1']
    atab, btab, ph8 = _tc_c(
        acc1, h1, degb, pr['sage1_Wr'], pr['sage1_b'][None],
        pr['nh_W1'], pr['nh_b1'][None],
        jnp.pad(pr['nh_W2'], ((0, 0), (0, 127))),
        jnp.pad(pr['nh_b2'], (0, 127))[None],
        ehW1[:128], ehW1[128:256], pr['eh_b1'][None])
    sa = _sc_edge_gather(atab, btab, sp, dp)
    ef = jnp.concatenate(
        [edge_static, q_obs[:, None], q_mask[:, None].astype(f32)], axis=1)
    ef = jnp.pad(ef, ((0, E_PAD - E), (0, 32 - ef.shape[1])))
    ehW1c = jnp.pad(ehW1[256:], ((0, 32 - (ehW1.shape[0] - 256)), (0, 0)))
    q8 = _tc_d(sa, ef, ehW1c,
               jnp.pad(pr['eh_W2'], ((0, 0), (0, 127))),
               jnp.pad(pr['eh_b2'], (0, 127))[None])

    return (ph8[:N, 0], q8[:E, 0])


# speedup vs baseline: 1.1699x; 1.1699x over previous
import functools

import jax
import jax.numpy as jnp
from jax import lax
from jax.experimental import pallas as pl
from jax.experimental.pallas import tpu as pltpu
from jax.experimental.pallas import tpu_sc as plsc

N = 10000
NP = 10240
E = 320000
EB = 2 * E
NW = 32
G_BI = 160
EB_PAD = NW * G_BI * 128
G_E = 80
E_PAD = NW * G_E * 128
NS = 16
RPT = NP // NS
NHR = NP // 128
HALF = NP // 2

f32 = jnp.float32


BLK = 256
BLK_E = 512


def _tca_body(nf, w, b, wl, h0_ref, p0_ref):
    h0 = jnp.dot(nf[...], w[...], preferred_element_type=f32) + b[...]
    h0_ref[...] = h0
    p0_ref[...] = jnp.dot(h0, wl[...], preferred_element_type=f32)


_tc_a = pl.pallas_call(
    _tca_body,
    grid=(NP // BLK,),
    in_specs=[
        pl.BlockSpec((BLK, 144), lambda i: (i, 0)),
        pl.BlockSpec((144, 128), lambda i: (0, 0)),
        pl.BlockSpec((1, 128), lambda i: (0, 0)),
        pl.BlockSpec((128, 128), lambda i: (0, 0)),
    ],
    out_specs=[
        pl.BlockSpec((BLK, 128), lambda i: (i, 0)),
        pl.BlockSpec((BLK, 128), lambda i: (i, 0)),
    ],
    out_shape=[
        jax.ShapeDtypeStruct((NP, 128), f32),
        jax.ShapeDtypeStruct((NP, 128), f32),
    ],
)


def _deg_col(degacc_ref):
    d2 = degacc_ref[0, 0] + degacc_ref[1, 0]
    a_const = (lax.broadcasted_iota(jnp.int32, (BLK, 2), 0) // 128
               == lax.broadcasted_iota(jnp.int32, (BLK, 2), 1)).astype(f32)
    sel = (lax.broadcasted_iota(jnp.int32, (BLK, 128), 0) % 128
           == lax.broadcasted_iota(jnp.int32, (BLK, 128), 1)).astype(f32)
    spread = jnp.dot(a_const, d2, preferred_element_type=f32)
    deg = jnp.sum(spread * sel, axis=1, keepdims=True)
    return jnp.maximum(deg, 1.0)


def _tcb_body(acc_ref, degacc_ref, h0_ref, wr, b, wl1, h1_ref, p1_ref, degb_ref):
    acc = acc_ref[0] + acc_ref[1]
    deg = _deg_col(degacc_ref)
    h1 = jnp.maximum(
        acc / deg + jnp.dot(h0_ref[...], wr[...], preferred_element_type=f32)
        + b[...], 0.0)
    h1_ref[...] = h1
    p1_ref[...] = jnp.dot(h1, wl1[...], preferred_element_type=f32)
    degb_ref[...] = jnp.broadcast_to(deg, (BLK, 128))


_tc_b = pl.pallas_call(
    _tcb_body,
    grid=(NP // BLK,),
    in_specs=[
        pl.BlockSpec((2, BLK, 128), lambda i: (0, i, 0)),
        pl.BlockSpec((2, 1, 2, 128), lambda i: (0, i, 0, 0)),
        pl.BlockSpec((BLK, 128), lambda i: (i, 0)),
        pl.BlockSpec((128, 128), lambda i: (0, 0)),
        pl.BlockSpec((1, 128), lambda i: (0, 0)),
        pl.BlockSpec((128, 128), lambda i: (0, 0)),
    ],
    out_specs=[
        pl.BlockSpec((BLK, 128), lambda i: (i, 0)),
        pl.BlockSpec((BLK, 128), lambda i: (i, 0)),
        pl.BlockSpec((BLK, 128), lambda i: (i, 0)),
    ],
    out_shape=[
        jax.ShapeDtypeStruct((NP, 128), f32),
        jax.ShapeDtypeStruct((NP, 128), f32),
        jax.ShapeDtypeStruct((NP, 128), f32),
    ],
)


def _tcc_body(acc_ref, h1_ref, degb_ref, wr, b, nw1, nb1, nw2, nb2, wa, wb,
              eb1, atab_ref, btab_ref, ph_ref):
    acc = acc_ref[0] + acc_ref[1]
    h2 = (acc / degb_ref[...]
          + jnp.dot(h1_ref[...], wr[...], preferred_element_type=f32) + b[...])
    atab_ref[...] = jnp.dot(h2, wa[...], preferred_element_type=f32) + eb1[...]
    btab_ref[...] = jnp.dot(h2, wb[...], preferred_element_type=f32)
    t = jnp.maximum(jnp.dot(h2, nw1[...], preferred_element_type=f32) + nb1[...], 0.0)
    ph = jnp.dot(t, nw2[...], preferred_element_type=f32) + nb2[...]
    ph_ref[...] = ph[:, :8]


_tc_c = pl.pallas_call(
    _tcc_body,
    grid=(NP // BLK,),
    in_specs=[
        pl.BlockSpec((2, BLK, 128), lambda i: (0, i, 0)),
        pl.BlockSpec((BLK, 128), lambda i: (i, 0)),
        pl.BlockSpec((BLK, 128), lambda i: (i, 0)),
        pl.BlockSpec((128, 128), lambda i: (0, 0)),
        pl.BlockSpec((1, 128), lambda i: (0, 0)),
        pl.BlockSpec((128, 128), lambda i: (0, 0)),
        pl.BlockSpec((1, 128), lambda i: (0, 0)),
        pl.BlockSpec((128, 128), lambda i: (0, 0)),
        pl.BlockSpec((1, 128), lambda i: (0, 0)),
        pl.BlockSpec((128, 128), lambda i: (0, 0)),
        pl.BlockSpec((128, 128), lambda i: (0, 0)),
        pl.BlockSpec((1, 128), lambda i: (0, 0)),
    ],
    out_specs=[
        pl.BlockSpec((BLK, 128), lambda i: (i, 0)),
        pl.BlockSpec((BLK, 128), lambda i: (i, 0)),
        pl.BlockSpec((BLK, 8), lambda i: (i, 0)),
    ],
    out_shape=[
        jax.ShapeDtypeStruct((NP, 128), f32),
        jax.ShapeDtypeStruct((NP, 128), f32),
        jax.ShapeDtypeStruct((NP, 8), f32),
    ],
)


def _tcd_body(sa_ref, sb_ref, ef_ref, wc, w2, b2, q_ref):
    hidden = (sa_ref[...] + sb_ref[...]
              + jnp.dot(ef_ref[...], wc[...], preferred_element_type=f32))
    q = jnp.dot(jnp.maximum(hidden, 0.0), w2[...], preferred_element_type=f32) + b2[...]
    q_ref[...] = q[:, :8]


_tc_d = pl.pallas_call(
    _tcd_body,
    grid=(E_PAD // BLK_E,),
    in_specs=[
        pl.BlockSpec((BLK_E, 128), lambda i: (i, 0)),
        pl.BlockSpec((BLK_E, 128), lambda i: (i, 0)),
        pl.BlockSpec((BLK_E, 32), lambda i: (i, 0)),
        pl.BlockSpec((32, 128), lambda i: (0, 0)),
        pl.BlockSpec((128, 128), lambda i: (0, 0)),
        pl.BlockSpec((1, 128), lambda i: (0, 0)),
    ],
    out_specs=pl.BlockSpec((BLK_E, 8), lambda i: (i, 0)),
    out_shape=jax.ShapeDtypeStruct((E_PAD, 8), f32),
)


_MESH = dict(core_axis_name="c", subcore_axis_name="s")


ACC_RPT = NP // NS
CH = 64
IB2 = 32
DEPTH = 4


def _make_sc_scatter(G2):
    @functools.partial(
        pl.kernel,
        out_type=jax.ShapeDtypeStruct((2, NP, 128), f32),
        mesh=plsc.VectorSubcoreMesh(**_MESH),
        scratch_types=(
            [pltpu.VMEM((IB2, CH), jnp.int32),
             pltpu.VMEM((IB2, CH), jnp.int32)]
            + [pltpu.VMEM((CH, 128), f32)] * DEPTH
            + [pltpu.VMEM_SHARED((NP, 128), f32)]
            + [pltpu.SemaphoreType.DMA] * (2 * DEPTH)
        ),
    )
    def sc_scatter(table, sidx, didx, zeros, out, sidx_v, didx_v, *scr):
        rbufs = scr[:DEPTH]
        acc_sh = scr[DEPTH]
        gsems = scr[DEPTH + 1:DEPTH + 1 + DEPTH]
        ssems = scr[DEPTH + 1 + DEPTH:]
        c = lax.axis_index("c")
        s = lax.axis_index("s")
        wid = s * 2 + c
        pltpu.sync_copy(zeros.at[pl.ds(s * ACC_RPT, ACC_RPT)],
                        acc_sh.at[pl.ds(s * ACC_RPT, ACC_RPT)])
        plsc.subcore_barrier()

        def blk_body(j, carry):
            pltpu.sync_copy(sidx.at[wid, pl.ds(j * IB2, IB2)], sidx_v)
            pltpu.sync_copy(didx.at[wid, pl.ds(j * IB2, IB2)], didx_v)

            def body(i, carry2):
                g = i * DEPTH
                hs = [pltpu.async_copy(table.at[sidx_v.at[g + k]], rbufs[k],
                                       gsems[k])
                      for k in range(DEPTH)]
                ws = []
                for k in range(DEPTH):
                    hs[k].wait()
                    ws.append(pltpu.async_copy(
                        rbufs[k], acc_sh.at[didx_v.at[g + k]], ssems[k],
                        add=True))
                for w in ws:
                    w.wait()
                return carry2

            lax.fori_loop(0, IB2 // DEPTH, body, 0)
            return carry

        lax.fori_loop(0, G2 // IB2, blk_body, 0)
        plsc.subcore_barrier()
        pltpu.sync_copy(acc_sh.at[pl.ds(s * ACC_RPT, ACC_RPT)],
                        out.at[c, pl.ds(s * ACC_RPT, ACC_RPT)])

    return sc_scatter


_sc_scatter = _make_sc_scatter(G_BI * 128 // CH)


@functools.partial(
    pl.kernel,
    out_type=jax.ShapeDtypeStruct((2, NHR, 128), f32),
    mesh=plsc.VectorSubcoreMesh(**_MESH),
    compiler_params=pltpu.CompilerParams(needs_layout_passes=False),
    scratch_types=[
        pltpu.VMEM((G_BI, 128), jnp.int32),
        pltpu.VMEM((16 * HALF,), f32),
        pltpu.VMEM((NHR, 128), f32),
        pltpu.VMEM((1, NHR), jnp.int32),
        pltpu.VMEM_SHARED((NHR, 128), f32),
        pltpu.SemaphoreType.DMA,
    ],
)
def _sc_deg(didx, rowids, zeros, out, didx_v, hist_v, deg_v, rid_v, acc_sh, sem):
    c = lax.axis_index("c")
    s = lax.axis_index("s")
    wid = s * 2 + c
    pltpu.sync_copy(didx.at[wid], didx_v)
    pltpu.sync_copy(rowids, rid_v)

    @pl.when(s < NHR // 8)
    def _zero():
        pltpu.sync_copy(zeros.at[pl.ds(s * 8, 8)], acc_sh.at[pl.ds(s * 8, 8)])

    lane_off = lax.iota(jnp.int32, 16) * HALF
    ones16 = jnp.ones((16,), f32)
    zeros16 = jnp.zeros((16,), f32)

    for half in range(2):
        lo = half * HALF

        def zero_body(j, carry):
            hist_v[pl.ds(j * 16, 16)] = zeros16
            return carry

        lax.fori_loop(0, 16 * (HALF // 16), zero_body, 0)

        def count_body(g, carry):
            for k in range(8):
                idx16 = didx_v[g, pl.ds(k * 16, 16)]
                m = (idx16 >= lo) & (idx16 < lo + HALF)
                idxc = jnp.where(m, idx16 - lo, 0) + lane_off
                plsc.addupdate_scatter(hist_v, [idxc], ones16, mask=m)
            return carry

        lax.fori_loop(0, G_BI, count_body, 0)

        def fold_body(j, carry):
            t = hist_v[pl.ds(j * 16, 16)]
            for r in range(1, 16):
                t = t + hist_v[pl.ds(r * HALF + j * 16, 16)]
            flat = half * HALF + j * 16
            deg_v[flat // 128, pl.ds(flat % 128, 16)] = t
            return carry

        lax.fori_loop(0, HALF // 16, fold_body, 0)

    plsc.subcore_barrier()
    pltpu.sync_copy(deg_v, acc_sh.at[rid_v.at[0]], add=True)
    plsc.subcore_barrier()

    @pl.when(s < NHR // 8)
    def _writeback():
        pltpu.sync_copy(acc_sh.at[pl.ds(s * 8, 8)], out.at[c, pl.ds(s * 8, 8)])


@functools.partial(
    pl.kernel,
    out_type=(
        jax.ShapeDtypeStruct((E_PAD, 128), f32),
        jax.ShapeDtypeStruct((E_PAD, 128), f32),
    ),
    mesh=plsc.VectorSubcoreMesh(**_MESH),
    scratch_types=[
        pltpu.VMEM((G_E, 128), jnp.int32),
        pltpu.VMEM((G_E, 128), jnp.int32),
        pltpu.VMEM((128, 128), f32),
        pltpu.VMEM((128, 128), f32),
        pltpu.VMEM((128, 128), f32),
        pltpu.VMEM((128, 128), f32),
        pltpu.SemaphoreType.DMA,
        pltpu.SemaphoreType.DMA,
        pltpu.SemaphoreType.DMA,
        pltpu.SemaphoreType.DMA,
        pltpu.SemaphoreType.DMA,
        pltpu.SemaphoreType.DMA,
        pltpu.SemaphoreType.DMA,
        pltpu.SemaphoreType.DMA,
    ],
)
def _sc_edge_gather(atab, btab, sidx, didx, outa, outb, sidx_v, didx_v,
                    ra0, rb0, ra1, rb1, ga0, gb0, ga1, gb1, wa0, wb0, wa1, wb1):
    c = lax.axis_index("c")
    s = lax.axis_index("s")
    wid = s * 2 + c
    pltpu.sync_copy(sidx.at[wid], sidx_v)
    pltpu.sync_copy(didx.at[wid], didx_v)
    base = wid * (G_E * 128)

    def body(i, carry):
        g = i * 2
        ha0 = pltpu.async_copy(atab.at[sidx_v.at[g]], ra0, ga0)
        hb0 = pltpu.async_copy(btab.at[didx_v.at[g]], rb0, gb0)
        ha1 = pltpu.async_copy(atab.at[sidx_v.at[g + 1]], ra1, ga1)
        hb1 = pltpu.async_copy(btab.at[didx_v.at[g + 1]], rb1, gb1)
        ha0.wait()
        va0 = pltpu.async_copy(ra0, outa.at[pl.ds(base + g * 128, 128)], wa0)
        hb0.wait()
        vb0 = pltpu.async_copy(rb0, outb.at[pl.ds(base + g * 128, 128)], wb0)
        ha1.wait()
        va1 = pltpu.async_copy(ra1, outa.at[pl.ds(base + (g + 1) * 128, 128)], wa1)
        hb1.wait()
        vb1 = pltpu.async_copy(rb1, outb.at[pl.ds(base + (g + 1) * 128, 128)], wb1)
        va0.wait()
        vb0.wait()
        va1.wait()
        vb1.wait()
        return carry

    lax.fori_loop(0, G_E // 2, body, 0)


def kernel(edge_index, node_static, edge_static, p_obs, q_obs, p_mask, q_mask, params):
    pr = params
    ei = edge_index.astype(jnp.int32)
    e0, e1 = ei[0], ei[1]

    nf = jnp.concatenate(
        [node_static, p_obs[:, None], p_mask[:, None].astype(f32)], axis=1)
    nf = jnp.pad(nf, ((0, NP - N), (0, 144 - nf.shape[1])))
    encW = jnp.pad(pr['enc_W'], ((0, 144 - pr['enc_W'].shape[0]), (0, 0)))

    src = jnp.concatenate([e0, e1])
    dst = jnp.concatenate([e1, e0])
    pad_bi = jnp.full((EB_PAD - EB,), N, jnp.int32)
    src_pad = jnp.concatenate([src, pad_bi])
    dst_pad = jnp.concatenate([dst, pad_bi])
    srcp = src_pad.reshape(NW, G_BI * 128 // CH, CH)
    dstp = dst_pad.reshape(NW, G_BI * 128 // CH, CH)
    pad_e = jnp.full((E_PAD - E,), N, jnp.int32)
    sp = jnp.concatenate([e0, pad_e]).reshape(NW, G_E, 128)
    dp = jnp.concatenate([e1, pad_e]).reshape(NW, G_E, 128)

    zeros_acc = jnp.zeros((NP, 128), f32)
    zeros_deg = jnp.zeros((NHR, 128), f32)
    rowids = jnp.arange(NHR, dtype=jnp.int32)[None, :]

    degacc = _sc_deg(dst_pad.reshape(NW, G_BI, 128), rowids, zeros_deg)
    h0, p0 = _tc_a(nf, encW, pr['enc_b'][None], pr['sage0_Wl'])
    acc0 = _sc_scatter(p0, srcp, dstp, zeros_acc)
    h1, p1, degb = _tc_b(acc0, degacc.reshape(2, NHR // 2, 2, 128), h0,
                         pr['sage0_Wr'], pr['sage0_b'][None], pr['sage1_Wl'])
    acc1 = _sc_scatter(p1, srcp, dstp, zeros_acc)
    ehW1 = pr['eh_W1']
    atab, btab, ph8 = _tc_c(
        acc1, h1, degb, pr['sage1_Wr'], pr['sage1_b'][None],
        pr['nh_W1'], pr['nh_b1'][None],
        jnp.pad(pr['nh_W2'], ((0, 0), (0, 127))),
        jnp.pad(pr['nh_b2'], (0, 127))[None],
        ehW1[:128], ehW1[128:256], pr['eh_b1'][None])
    sa, sb = _sc_edge_gather(atab, btab, sp, dp)
    ef = jnp.concatenate(
        [edge_static, q_obs[:, None], q_mask[:, None].astype(f32)], axis=1)
    ef = jnp.pad(ef, ((0, E_PAD - E), (0, 32 - ef.shape[1])))
    ehW1c = jnp.pad(ehW1[256:], ((0, 32 - (ehW1.shape[0] - 256)), (0, 0)))
    q8 = _tc_d(sa, sb, ef, ehW1c,
               jnp.pad(pr['eh_W2'], ((0, 0), (0, 127))),
               jnp.pad(pr['eh_b2'], (0, 127))[None])

    return (ph8[:N, 0], q8[:E, 0])


# speedup vs baseline: 1.2599x; 1.0770x over previous
import functools

import jax
import jax.numpy as jnp
from jax import lax
from jax.experimental import pallas as pl
from jax.experimental.pallas import tpu as pltpu
from jax.experimental.pallas import tpu_sc as plsc

N = 10000
NP = 10240
E = 320000
EB = 2 * E
NW = 32
G_BI = 160
EB_PAD = NW * G_BI * 128
G_E = 80
E_PAD = NW * G_E * 128
NS = 16
RPT = NP // NS
NHR = NP // 128
HALF = NP // 2

f32 = jnp.float32


BLK = 256
BLK_E = 512


def _tca_body(nf, w, b, wl, h0_ref, p0_ref):
    h0 = jnp.dot(nf[...], w[...], preferred_element_type=f32) + b[...]
    h0_ref[...] = h0
    p0_ref[...] = jnp.dot(h0, wl[...], preferred_element_type=f32)


_tc_a = pl.pallas_call(
    _tca_body,
    grid=(NP // BLK,),
    in_specs=[
        pl.BlockSpec((BLK, 144), lambda i: (i, 0)),
        pl.BlockSpec((144, 128), lambda i: (0, 0)),
        pl.BlockSpec((1, 128), lambda i: (0, 0)),
        pl.BlockSpec((128, 128), lambda i: (0, 0)),
    ],
    out_specs=[
        pl.BlockSpec((BLK, 128), lambda i: (i, 0)),
        pl.BlockSpec((BLK, 128), lambda i: (i, 0)),
    ],
    out_shape=[
        jax.ShapeDtypeStruct((NP, 128), f32),
        jax.ShapeDtypeStruct((NP, 128), f32),
    ],
)


def _deg_col(degacc_ref):
    d2 = degacc_ref[0, 0] + degacc_ref[1, 0]
    a_const = (lax.broadcasted_iota(jnp.int32, (BLK, 2), 0) // 128
               == lax.broadcasted_iota(jnp.int32, (BLK, 2), 1)).astype(f32)
    sel = (lax.broadcasted_iota(jnp.int32, (BLK, 128), 0) % 128
           == lax.broadcasted_iota(jnp.int32, (BLK, 128), 1)).astype(f32)
    spread = jnp.dot(a_const, d2, preferred_element_type=f32)
    deg = jnp.sum(spread * sel, axis=1, keepdims=True)
    return jnp.maximum(deg, 1.0)


def _tcb_body(acc_ref, degacc_ref, h0_ref, wr, b, wl1, h1_ref, p1_ref, degb_ref):
    acc = acc_ref[0] + acc_ref[1]
    deg = _deg_col(degacc_ref)
    h1 = jnp.maximum(
        acc / deg + jnp.dot(h0_ref[...], wr[...], preferred_element_type=f32)
        + b[...], 0.0)
    h1_ref[...] = h1
    p1_ref[...] = jnp.dot(h1, wl1[...], preferred_element_type=f32)
    degb_ref[...] = jnp.broadcast_to(deg, (BLK, 128))


_tc_b = pl.pallas_call(
    _tcb_body,
    grid=(NP // BLK,),
    in_specs=[
        pl.BlockSpec((2, BLK, 128), lambda i: (0, i, 0)),
        pl.BlockSpec((2, 1, 2, 128), lambda i: (0, i, 0, 0)),
        pl.BlockSpec((BLK, 128), lambda i: (i, 0)),
        pl.BlockSpec((128, 128), lambda i: (0, 0)),
        pl.BlockSpec((1, 128), lambda i: (0, 0)),
        pl.BlockSpec((128, 128), lambda i: (0, 0)),
    ],
    out_specs=[
        pl.BlockSpec((BLK, 128), lambda i: (i, 0)),
        pl.BlockSpec((BLK, 128), lambda i: (i, 0)),
        pl.BlockSpec((BLK, 128), lambda i: (i, 0)),
    ],
    out_shape=[
        jax.ShapeDtypeStruct((NP, 128), f32),
        jax.ShapeDtypeStruct((NP, 128), f32),
        jax.ShapeDtypeStruct((NP, 128), f32),
    ],
)


def _tcc_body(acc_ref, h1_ref, degb_ref, wr, b, nw1, nb1, nw2, nb2, wa, wb,
              eb1, atab_ref, btab_ref, ph_ref):
    acc = acc_ref[0] + acc_ref[1]
    h2 = (acc / degb_ref[...]
          + jnp.dot(h1_ref[...], wr[...], preferred_element_type=f32) + b[...])
    atab_ref[...] = jnp.dot(h2, wa[...], preferred_element_type=f32) + eb1[...]
    btab_ref[...] = jnp.dot(h2, wb[...], preferred_element_type=f32)
    t = jnp.maximum(jnp.dot(h2, nw1[...], preferred_element_type=f32) + nb1[...], 0.0)
    ph = jnp.dot(t, nw2[...], preferred_element_type=f32) + nb2[...]
    ph_ref[...] = ph[:, :8]


_tc_c = pl.pallas_call(
    _tcc_body,
    grid=(NP // BLK,),
    in_specs=[
        pl.BlockSpec((2, BLK, 128), lambda i: (0, i, 0)),
        pl.BlockSpec((BLK, 128), lambda i: (i, 0)),
        pl.BlockSpec((BLK, 128), lambda i: (i, 0)),
        pl.BlockSpec((128, 128), lambda i: (0, 0)),
        pl.BlockSpec((1, 128), lambda i: (0, 0)),
        pl.BlockSpec((128, 128), lambda i: (0, 0)),
        pl.BlockSpec((1, 128), lambda i: (0, 0)),
        pl.BlockSpec((128, 128), lambda i: (0, 0)),
        pl.BlockSpec((1, 128), lambda i: (0, 0)),
        pl.BlockSpec((128, 128), lambda i: (0, 0)),
        pl.BlockSpec((128, 128), lambda i: (0, 0)),
        pl.BlockSpec((1, 128), lambda i: (0, 0)),
    ],
    out_specs=[
        pl.BlockSpec((BLK, 128), lambda i: (i, 0)),
        pl.BlockSpec((BLK, 128), lambda i: (i, 0)),
        pl.BlockSpec((BLK, 8), lambda i: (i, 0)),
    ],
    out_shape=[
        jax.ShapeDtypeStruct((NP, 128), f32),
        jax.ShapeDtypeStruct((NP, 128), f32),
        jax.ShapeDtypeStruct((NP, 8), f32),
    ],
)


def _tcd_body(sa_ref, sb_ref, ef_ref, wc, w2, b2, q_ref):
    hidden = (sa_ref[...] + sb_ref[...]
              + jnp.dot(ef_ref[...], wc[...], preferred_element_type=f32))
    q = jnp.dot(jnp.maximum(hidden, 0.0), w2[...], preferred_element_type=f32) + b2[...]
    q_ref[...] = q[:, :8]


_tc_d = pl.pallas_call(
    _tcd_body,
    grid=(E_PAD // BLK_E,),
    in_specs=[
        pl.BlockSpec((BLK_E, 128), lambda i: (i, 0)),
        pl.BlockSpec((BLK_E, 128), lambda i: (i, 0)),
        pl.BlockSpec((BLK_E, 32), lambda i: (i, 0)),
        pl.BlockSpec((32, 128), lambda i: (0, 0)),
        pl.BlockSpec((128, 128), lambda i: (0, 0)),
        pl.BlockSpec((1, 128), lambda i: (0, 0)),
    ],
    out_specs=pl.BlockSpec((BLK_E, 8), lambda i: (i, 0)),
    out_shape=jax.ShapeDtypeStruct((E_PAD, 8), f32),
)


_MESH = dict(core_axis_name="c", subcore_axis_name="s")


ACC_RPT = NP // NS
CH = 32
IB2 = 32
DEPTH = 8


def _make_sc_scatter(G2):
    @functools.partial(
        pl.kernel,
        out_type=jax.ShapeDtypeStruct((2, NP, 128), f32),
        mesh=plsc.VectorSubcoreMesh(**_MESH),
        scratch_types=(
            [pltpu.VMEM((IB2, CH), jnp.int32),
             pltpu.VMEM((IB2, CH), jnp.int32)]
            + [pltpu.VMEM((CH, 128), f32)] * DEPTH
            + [pltpu.VMEM_SHARED((NP, 128), f32)]
            + [pltpu.SemaphoreType.DMA] * (2 * DEPTH)
        ),
    )
    def sc_scatter(table, sidx, didx, zeros, out, sidx_v, didx_v, *scr):
        rbufs = scr[:DEPTH]
        acc_sh = scr[DEPTH]
        gsems = scr[DEPTH + 1:DEPTH + 1 + DEPTH]
        ssems = scr[DEPTH + 1 + DEPTH:]
        c = lax.axis_index("c")
        s = lax.axis_index("s")
        wid = s * 2 + c
        pltpu.sync_copy(zeros.at[pl.ds(s * ACC_RPT, ACC_RPT)],
                        acc_sh.at[pl.ds(s * ACC_RPT, ACC_RPT)])
        plsc.subcore_barrier()

        def blk_body(j, carry):
            pltpu.sync_copy(sidx.at[wid, pl.ds(j * IB2, IB2)], sidx_v)
            pltpu.sync_copy(didx.at[wid, pl.ds(j * IB2, IB2)], didx_v)

            def body(i, carry2):
                g = i * DEPTH
                hs = [pltpu.async_copy(table.at[sidx_v.at[g + k]], rbufs[k],
                                       gsems[k])
                      for k in range(DEPTH)]
                ws = []
                for k in range(DEPTH):
                    hs[k].wait()
                    ws.append(pltpu.async_copy(
                        rbufs[k], acc_sh.at[didx_v.at[g + k]], ssems[k],
                        add=True))
                for w in ws:
                    w.wait()
                return carry2

            lax.fori_loop(0, IB2 // DEPTH, body, 0)
            return carry

        lax.fori_loop(0, G2 // IB2, blk_body, 0)
        plsc.subcore_barrier()
        pltpu.sync_copy(acc_sh.at[pl.ds(s * ACC_RPT, ACC_RPT)],
                        out.at[c, pl.ds(s * ACC_RPT, ACC_RPT)])

    return sc_scatter


_sc_scatter = _make_sc_scatter(G_BI * 128 // CH)


@functools.partial(
    pl.kernel,
    out_type=jax.ShapeDtypeStruct((2, NHR, 128), f32),
    mesh=plsc.VectorSubcoreMesh(**_MESH),
    compiler_params=pltpu.CompilerParams(needs_layout_passes=False),
    scratch_types=[
        pltpu.VMEM((G_BI, 128), jnp.int32),
        pltpu.VMEM((16 * HALF,), f32),
        pltpu.VMEM((NHR, 128), f32),
        pltpu.VMEM((1, NHR), jnp.int32),
        pltpu.VMEM_SHARED((NHR, 128), f32),
        pltpu.SemaphoreType.DMA,
    ],
)
def _sc_deg(didx, rowids, zeros, out, didx_v, hist_v, deg_v, rid_v, acc_sh, sem):
    c = lax.axis_index("c")
    s = lax.axis_index("s")
    wid = s * 2 + c
    pltpu.sync_copy(didx.at[wid], didx_v)
    pltpu.sync_copy(rowids, rid_v)

    @pl.when(s < NHR // 8)
    def _zero():
        pltpu.sync_copy(zeros.at[pl.ds(s * 8, 8)], acc_sh.at[pl.ds(s * 8, 8)])

    lane_off = lax.iota(jnp.int32, 16) * HALF
    ones16 = jnp.ones((16,), f32)
    zeros16 = jnp.zeros((16,), f32)

    for half in range(2):
        lo = half * HALF

        def zero_body(j, carry):
            hist_v[pl.ds(j * 16, 16)] = zeros16
            return carry

        lax.fori_loop(0, 16 * (HALF // 16), zero_body, 0)

        def count_body(g, carry):
            for k in range(8):
                idx16 = didx_v[g, pl.ds(k * 16, 16)]
                m = (idx16 >= lo) & (idx16 < lo + HALF)
                idxc = jnp.where(m, idx16 - lo, 0) + lane_off
                plsc.addupdate_scatter(hist_v, [idxc], ones16, mask=m)
            return carry

        lax.fori_loop(0, G_BI, count_body, 0)

        def fold_body(j, carry):
            t = hist_v[pl.ds(j * 16, 16)]
            for r in range(1, 16):
                t = t + hist_v[pl.ds(r * HALF + j * 16, 16)]
            flat = half * HALF + j * 16
            deg_v[flat // 128, pl.ds(flat % 128, 16)] = t
            return carry

        lax.fori_loop(0, HALF // 16, fold_body, 0)

    plsc.subcore_barrier()
    pltpu.sync_copy(deg_v, acc_sh.at[rid_v.at[0]], add=True)
    plsc.subcore_barrier()

    @pl.when(s < NHR // 8)
    def _writeback():
        pltpu.sync_copy(acc_sh.at[pl.ds(s * 8, 8)], out.at[c, pl.ds(s * 8, 8)])


@functools.partial(
    pl.kernel,
    out_type=(
        jax.ShapeDtypeStruct((E_PAD, 128), f32),
        jax.ShapeDtypeStruct((E_PAD, 128), f32),
    ),
    mesh=plsc.VectorSubcoreMesh(**_MESH),
    scratch_types=[
        pltpu.VMEM((G_E, 128), jnp.int32),
        pltpu.VMEM((G_E, 128), jnp.int32),
        pltpu.VMEM((128, 128), f32),
        pltpu.VMEM((128, 128), f32),
        pltpu.VMEM((128, 128), f32),
        pltpu.VMEM((128, 128), f32),
        pltpu.SemaphoreType.DMA,
        pltpu.SemaphoreType.DMA,
        pltpu.SemaphoreType.DMA,
        pltpu.SemaphoreType.DMA,
        pltpu.SemaphoreType.DMA,
        pltpu.SemaphoreType.DMA,
        pltpu.SemaphoreType.DMA,
        pltpu.SemaphoreType.DMA,
    ],
)
def _sc_edge_gather(atab, btab, sidx, didx, outa, outb, sidx_v, didx_v,
                    ra0, rb0, ra1, rb1, ga0, gb0, ga1, gb1, wa0, wb0, wa1, wb1):
    c = lax.axis_index("c")
    s = lax.axis_index("s")
    wid = s * 2 + c
    pltpu.sync_copy(sidx.at[wid], sidx_v)
    pltpu.sync_copy(didx.at[wid], didx_v)
    base = wid * (G_E * 128)

    def body(i, carry):
        g = i * 2
        ha0 = pltpu.async_copy(atab.at[sidx_v.at[g]], ra0, ga0)
        hb0 = pltpu.async_copy(btab.at[didx_v.at[g]], rb0, gb0)
        ha1 = pltpu.async_copy(atab.at[sidx_v.at[g + 1]], ra1, ga1)
        hb1 = pltpu.async_copy(btab.at[didx_v.at[g + 1]], rb1, gb1)
        ha0.wait()
        va0 = pltpu.async_copy(ra0, outa.at[pl.ds(base + g * 128, 128)], wa0)
        hb0.wait()
        vb0 = pltpu.async_copy(rb0, outb.at[pl.ds(base + g * 128, 128)], wb0)
        ha1.wait()
        va1 = pltpu.async_copy(ra1, outa.at[pl.ds(base + (g + 1) * 128, 128)], wa1)
        hb1.wait()
        vb1 = pltpu.async_copy(rb1, outb.at[pl.ds(base + (g + 1) * 128, 128)], wb1)
        va0.wait()
        vb0.wait()
        va1.wait()
        vb1.wait()
        return carry

    lax.fori_loop(0, G_E // 2, body, 0)


def kernel(edge_index, node_static, edge_static, p_obs, q_obs, p_mask, q_mask, params):
    pr = params
    ei = edge_index.astype(jnp.int32)
    e0, e1 = ei[0], ei[1]

    nf = jnp.concatenate(
        [node_static, p_obs[:, None], p_mask[:, None].astype(f32)], axis=1)
    nf = jnp.pad(nf, ((0, NP - N), (0, 144 - nf.shape[1])))
    encW = jnp.pad(pr['enc_W'], ((0, 144 - pr['enc_W'].shape[0]), (0, 0)))

    src = jnp.concatenate([e0, e1])
    dst = jnp.concatenate([e1, e0])
    pad_bi = jnp.full((EB_PAD - EB,), N, jnp.int32)
    src_pad = jnp.concatenate([src, pad_bi])
    dst_pad = jnp.concatenate([dst, pad_bi])
    srcp = src_pad.reshape(NW, G_BI * 128 // CH, CH)
    dstp = dst_pad.reshape(NW, G_BI * 128 // CH, CH)
    pad_e = jnp.full((E_PAD - E,), N, jnp.int32)
    sp = jnp.concatenate([e0, pad_e]).reshape(NW, G_E, 128)
    dp = jnp.concatenate([e1, pad_e]).reshape(NW, G_E, 128)

    zeros_acc = jnp.zeros((NP, 128), f32)
    zeros_deg = jnp.zeros((NHR, 128), f32)
    rowids = jnp.arange(NHR, dtype=jnp.int32)[None, :]

    degacc = _sc_deg(dst_pad.reshape(NW, G_BI, 128), rowids, zeros_deg)
    h0, p0 = _tc_a(nf, encW, pr['enc_b'][None], pr['sage0_Wl'])
    acc0 = _sc_scatter(p0, srcp, dstp, zeros_acc)
    h1, p1, degb = _tc_b(acc0, degacc.reshape(2, NHR // 2, 2, 128), h0,
                         pr['sage0_Wr'], pr['sage0_b'][None], pr['sage1_Wl'])
    acc1 = _sc_scatter(p1, srcp, dstp, zeros_acc)
    ehW1 = pr['eh_W1']
    atab, btab, ph8 = _tc_c(
        acc1, h1, degb, pr['sage1_Wr'], pr['sage1_b'][None],
        pr['nh_W1'], pr['nh_b1'][None],
        jnp.pad(pr['nh_W2'], ((0, 0), (0, 127))),
        jnp.pad(pr['nh_b2'], (0, 127))[None],
        ehW1[:128], ehW1[128:256], pr['eh_b1'][None])
    sa, sb = _sc_edge_gather(atab, btab, sp, dp)
    ef = jnp.concatenate(
        [edge_static, q_obs[:, None], q_mask[:, None].astype(f32)], axis=1)
    ef = jnp.pad(ef, ((0, E_PAD - E), (0, 32 - ef.shape[1])))
    ehW1c = jnp.pad(ehW1[256:], ((0, 32 - (ehW1.shape[0] - 256)), (0, 0)))
    q8 = _tc_d(sa, sb, ef, ehW1c,
               jnp.pad(pr['eh_W2'], ((0, 0), (0, 127))),
               jnp.pad(pr['eh_b2'], (0, 127))[None])

    return (ph8[:N, 0], q8[:E, 0])
